# Initial kernel scaffold; baseline (speedup 1.0000x reference)
import jax
import jax.numpy as jnp
from jax.experimental import pallas as pl


def kernel(x, edge_index, edge_weight, W1, b1, W2, b2):
    raise NotImplementedError("write your pallas kernel here")


# speedup vs baseline: 5.3828x; 5.3828x over previous
import dataclasses
import functools

import jax
import jax.numpy as jnp
from jax import lax
from jax.experimental import pallas as pl
from jax.experimental.pallas import tpu as pltpu
from jax.experimental.pallas import tpu_sc as plsc

N_NODES = 10000
N_EDGES = 320000
D_FEAT = 128
HIDDEN = 64
DL = 16
N_LABELS = 3

NC = 2
NS = 16
NW = NC * NS
L = 16
EC = 128
NCHUNK = N_EDGES // EC
RB = 80
NRB = N_NODES // RB

BM = 2000


def _sc_agg(h, src, dst, w, D):
    mesh = plsc.VectorSubcoreMesh(core_axis_name="c", subcore_axis_name="s")
    cp = pltpu.CompilerParams(use_tc_tiling_on_sc=False)
    if "needs_layout_passes" in pltpu.CompilerParams.__dataclass_fields__:
        cp = dataclasses.replace(cp, needs_layout_passes=False)

    @functools.partial(
        pl.kernel,
        mesh=mesh,
        compiler_params=cp,
        out_type=jax.ShapeDtypeStruct((NC, N_NODES, D), jnp.float32),
        scratch_types=[
            pltpu.VMEM((EC,), jnp.int32),
            pltpu.VMEM((EC,), jnp.int32),
            pltpu.VMEM((EC,), jnp.float32),
            pltpu.VMEM((EC, D), jnp.float32),
            pltpu.VMEM((RB, D), jnp.float32),
            pltpu.VMEM_SHARED((N_NODES, D), jnp.float32),
        ],
    )
    def k(h_hbm, src_hbm, dst_hbm, w_hbm, out_hbm,
          src_v, dst_v, w_v, rows_v, stage_v, acc_sh):
        cid = lax.axis_index("c")
        sid = lax.axis_index("s")
        wid = sid * NC + cid

        zvec = jnp.zeros((L,), jnp.float32)

        @pl.loop(0, RB)
        def _(i):
            for j in range(D // L):
                stage_v[i, pl.ds(j * L, L)] = zvec

        @pl.loop(sid, NRB, step=NS)
        def _(b):
            pltpu.sync_copy(stage_v, acc_sh.at[pl.ds(b * RB, RB)])

        plsc.subcore_barrier()

        @pl.loop(wid, NCHUNK, step=NW)
        def _(c):
            base = c * EC
            pltpu.sync_copy(src_hbm.at[pl.ds(base, EC)], src_v)
            pltpu.sync_copy(dst_hbm.at[pl.ds(base, EC)], dst_v)
            pltpu.sync_copy(w_hbm.at[pl.ds(base, EC)], w_v)
            pltpu.sync_copy(h_hbm.at[src_v], rows_v)

            @pl.loop(0, EC)
            def _(e):
                wb = plsc.load_gather(w_v, [jnp.full((L,), e, jnp.int32)])
                for j in range(D // L):
                    rows_v[e, pl.ds(j * L, L)] = rows_v[e, pl.ds(j * L, L)] * wb

            pltpu.sync_copy(rows_v, acc_sh.at[dst_v], add=True)

        plsc.subcore_barrier()

        @pl.loop(sid, NRB, step=NS)
        def _(b):
            pltpu.sync_copy(acc_sh.at[pl.ds(b * RB, RB)], stage_v)
            pltpu.sync_copy(stage_v, out_hbm.at[cid, pl.ds(b * RB, RB)])

    return k(h, src, dst, w)


def _mm_kernel(x_ref, w_ref, o_ref):
    o_ref[...] = jnp.dot(x_ref[...], w_ref[...],
                         preferred_element_type=jnp.float32,
                         precision=lax.Precision.HIGHEST)


def _tc_matmul(x, w):
    m, kdim = x.shape
    n = w.shape[1]
    return pl.pallas_call(
        _mm_kernel,
        grid=(m // BM,),
        in_specs=[pl.BlockSpec((BM, kdim), lambda i: (i, 0)),
                  pl.BlockSpec((kdim, n), lambda i: (0, 0))],
        out_specs=pl.BlockSpec((BM, n), lambda i: (i, 0)),
        out_shape=jax.ShapeDtypeStruct((m, n), jnp.float32),
    )(x, w)


def _mid_kernel(p0_ref, p1_ref, b1_ref, w2_ref, o_ref):
    h = jnp.maximum(p0_ref[...] + p1_ref[...] + b1_ref[...], 0.0)
    o_ref[...] = jnp.dot(h, w2_ref[...],
                         preferred_element_type=jnp.float32,
                         precision=lax.Precision.HIGHEST)


def _tc_mid(p0, p1, b1, w2p):
    m, kdim = p0.shape
    n = w2p.shape[1]
    return pl.pallas_call(
        _mid_kernel,
        grid=(m // BM,),
        in_specs=[pl.BlockSpec((BM, kdim), lambda i: (i, 0)),
                  pl.BlockSpec((BM, kdim), lambda i: (i, 0)),
                  pl.BlockSpec((1, kdim), lambda i: (0, 0)),
                  pl.BlockSpec((kdim, n), lambda i: (0, 0))],
        out_specs=pl.BlockSpec((BM, n), lambda i: (i, 0)),
        out_shape=jax.ShapeDtypeStruct((m, n), jnp.float32),
    )(p0, p1, b1, w2p)


def _sm_kernel(q0_ref, q1_ref, b2_ref, o_ref):
    s = q0_ref[...] + q1_ref[...] + b2_ref[...]
    col = lax.broadcasted_iota(jnp.int32, s.shape, 1)
    s = jnp.where(col < N_LABELS, s, -1e30)
    mx = jnp.max(s, axis=-1, keepdims=True)
    e = jnp.exp(s - mx)
    o_ref[...] = e / jnp.sum(e, axis=-1, keepdims=True)


def _tc_softmax(q0, q1, b2p):
    m, n = q0.shape
    return pl.pallas_call(
        _sm_kernel,
        grid=(m // BM,),
        in_specs=[pl.BlockSpec((BM, n), lambda i: (i, 0)),
                  pl.BlockSpec((BM, n), lambda i: (i, 0)),
                  pl.BlockSpec((1, n), lambda i: (0, 0))],
        out_specs=pl.BlockSpec((BM, n), lambda i: (i, 0)),
        out_shape=jax.ShapeDtypeStruct((m, n), jnp.float32),
    )(q0, q1, b2p)


def kernel(x, edge_index, edge_weight, W1, b1, W2, b2):
    src = edge_index[0].astype(jnp.int32)
    dst = edge_index[1].astype(jnp.int32)
    w = edge_weight.astype(jnp.float32)

    h1 = _tc_matmul(x, W1)
    p1 = _sc_agg(h1, src, dst, w, HIDDEN)

    w2p = jnp.pad(W2, ((0, 0), (0, DL - N_LABELS)))
    h2 = _tc_mid(p1[0], p1[1], b1.reshape(1, -1), w2p)
    p2 = _sc_agg(h2, src, dst, w, DL)

    b2p = jnp.pad(b2, (0, DL - N_LABELS)).reshape(1, -1)
    out = _tc_softmax(p2[0], p2[1], b2p)
    return out[:, :N_LABELS]


# speedup vs baseline: 8.3772x; 1.5563x over previous
import dataclasses
import functools

import jax
import jax.numpy as jnp
from jax import lax
from jax.experimental import pallas as pl
from jax.experimental.pallas import tpu as pltpu
from jax.experimental.pallas import tpu_sc as plsc

N_NODES = 10000
N_EDGES = 320000
D_FEAT = 128
HIDDEN = 64
DL = 16
N_LABELS = 3

NC = 2
NS = 16
NW = NC * NS
L = 16
EC = 128
EPT = N_EDGES // NW
NFC = EPT // EC
TAIL = EPT - NFC * EC
RB = 80
NRB = N_NODES // RB

BM = 2000


def _sc_agg(h, src, dst, w, D):
    mesh = plsc.VectorSubcoreMesh(core_axis_name="c", subcore_axis_name="s")
    cp = pltpu.CompilerParams(use_tc_tiling_on_sc=False)
    if "needs_layout_passes" in pltpu.CompilerParams.__dataclass_fields__:
        cp = dataclasses.replace(cp, needs_layout_passes=False)

    @functools.partial(
        pl.kernel,
        mesh=mesh,
        compiler_params=cp,
        out_type=jax.ShapeDtypeStruct((NC, N_NODES, D), jnp.float32),
        scratch_types=[
            pltpu.VMEM((2, EC), jnp.int32),
            pltpu.VMEM((2, EC), jnp.int32),
            pltpu.VMEM((2, EC), jnp.float32),
            pltpu.VMEM((2, EC, D), jnp.float32),
            pltpu.VMEM((RB, D), jnp.float32),
            pltpu.VMEM((TAIL,), jnp.int32),
            pltpu.VMEM((TAIL,), jnp.int32),
            pltpu.VMEM((TAIL,), jnp.float32),
            pltpu.VMEM((TAIL, D), jnp.float32),
            pltpu.VMEM_SHARED((N_NODES, D), jnp.float32),
            pltpu.SemaphoreType.DMA,
            pltpu.SemaphoreType.DMA,
        ],
    )
    def k(h_hbm, src_hbm, dst_hbm, w_hbm, out_hbm,
          src_v, dst_v, w_v, rows_v, stage_v,
          tsrc_v, tdst_v, tw_v, trows_v, acc_sh, gsem0, gsem1):
        cid = lax.axis_index("c")
        sid = lax.axis_index("s")
        wid = sid * NC + cid
        ebase = wid * EPT
        gsems = (gsem0, gsem1)

        zvec = jnp.zeros((L,), jnp.float32)

        @pl.loop(0, RB)
        def _(i):
            for j in range(D // L):
                stage_v[i, pl.ds(j * L, L)] = zvec

        @pl.loop(sid, NRB, step=NS)
        def _(b):
            pltpu.sync_copy(stage_v, acc_sh.at[pl.ds(b * RB, RB)])

        plsc.subcore_barrier()

        def load_idx(c, b):
            base = ebase + c * EC
            pltpu.sync_copy(src_hbm.at[pl.ds(base, EC)], src_v.at[b])
            pltpu.sync_copy(dst_hbm.at[pl.ds(base, EC)], dst_v.at[b])
            pltpu.sync_copy(w_hbm.at[pl.ds(base, EC)], w_v.at[b])

        def start_gather(b):
            pltpu.async_copy(h_hbm.at[src_v.at[b]], rows_v.at[b], gsems[b])

        def wait_gather(b):
            pltpu.make_async_copy(h_hbm.at[src_v.at[b]], rows_v.at[b],
                                  gsems[b]).wait()

        def scale_scatter(b):
            @plsc.parallel_loop(0, EC, unroll=8)
            def _(e):
                wb = plsc.load_gather(w_v.at[b], [jnp.full((L,), e, jnp.int32)])
                for j in range(D // L):
                    rows_v[b, e, pl.ds(j * L, L)] = (
                        rows_v[b, e, pl.ds(j * L, L)] * wb)

            pltpu.sync_copy(rows_v.at[b], acc_sh.at[dst_v.at[b]], add=True)

        load_idx(0, 0)
        start_gather(0)

        @pl.loop(0, NFC, step=2)
        def _(c):
            load_idx(c + 1, 1)
            start_gather(1)
            wait_gather(0)
            scale_scatter(0)

            @pl.when(c + 2 < NFC)
            def _():
                load_idx(c + 2, 0)
                start_gather(0)

            wait_gather(1)
            scale_scatter(1)

        tbase = ebase + NFC * EC
        pltpu.sync_copy(src_hbm.at[pl.ds(tbase, TAIL)], tsrc_v)
        pltpu.sync_copy(dst_hbm.at[pl.ds(tbase, TAIL)], tdst_v)
        pltpu.sync_copy(w_hbm.at[pl.ds(tbase, TAIL)], tw_v)
        pltpu.sync_copy(h_hbm.at[tsrc_v], trows_v)

        @pl.loop(0, TAIL)
        def _(e):
            wb = plsc.load_gather(tw_v, [jnp.full((L,), e, jnp.int32)])
            for j in range(D // L):
                trows_v[e, pl.ds(j * L, L)] = trows_v[e, pl.ds(j * L, L)] * wb

        pltpu.sync_copy(trows_v, acc_sh.at[tdst_v], add=True)

        plsc.subcore_barrier()

        @pl.loop(sid, NRB, step=NS)
        def _(b):
            pltpu.sync_copy(acc_sh.at[pl.ds(b * RB, RB)], stage_v)
            pltpu.sync_copy(stage_v, out_hbm.at[cid, pl.ds(b * RB, RB)])

    return k(h, src, dst, w)


def _mm_kernel(x_ref, w_ref, o_ref):
    o_ref[...] = jnp.dot(x_ref[...], w_ref[...],
                         preferred_element_type=jnp.float32,
                         precision=lax.Precision.HIGHEST)


def _tc_matmul(x, w):
    m, kdim = x.shape
    n = w.shape[1]
    return pl.pallas_call(
        _mm_kernel,
        grid=(m // BM,),
        in_specs=[pl.BlockSpec((BM, kdim), lambda i: (i, 0)),
                  pl.BlockSpec((kdim, n), lambda i: (0, 0))],
        out_specs=pl.BlockSpec((BM, n), lambda i: (i, 0)),
        out_shape=jax.ShapeDtypeStruct((m, n), jnp.float32),
    )(x, w)


def _mid_kernel(p0_ref, p1_ref, b1_ref, w2_ref, o_ref):
    h = jnp.maximum(p0_ref[...] + p1_ref[...] + b1_ref[...], 0.0)
    o_ref[...] = jnp.dot(h, w2_ref[...],
                         preferred_element_type=jnp.float32,
                         precision=lax.Precision.HIGHEST)


def _tc_mid(p0, p1, b1, w2p):
    m, kdim = p0.shape
    n = w2p.shape[1]
    return pl.pallas_call(
        _mid_kernel,
        grid=(m // BM,),
        in_specs=[pl.BlockSpec((BM, kdim), lambda i: (i, 0)),
                  pl.BlockSpec((BM, kdim), lambda i: (i, 0)),
                  pl.BlockSpec((1, kdim), lambda i: (0, 0)),
                  pl.BlockSpec((kdim, n), lambda i: (0, 0))],
        out_specs=pl.BlockSpec((BM, n), lambda i: (i, 0)),
        out_shape=jax.ShapeDtypeStruct((m, n), jnp.float32),
    )(p0, p1, b1, w2p)


def _sm_kernel(q0_ref, q1_ref, b2_ref, o_ref):
    s = q0_ref[...] + q1_ref[...] + b2_ref[...]
    col = lax.broadcasted_iota(jnp.int32, s.shape, 1)
    s = jnp.where(col < N_LABELS, s, -1e30)
    mx = jnp.max(s, axis=-1, keepdims=True)
    e = jnp.exp(s - mx)
    o_ref[...] = e / jnp.sum(e, axis=-1, keepdims=True)


def _tc_softmax(q0, q1, b2p):
    m, n = q0.shape
    return pl.pallas_call(
        _sm_kernel,
        grid=(m // BM,),
        in_specs=[pl.BlockSpec((BM, n), lambda i: (i, 0)),
                  pl.BlockSpec((BM, n), lambda i: (i, 0)),
                  pl.BlockSpec((1, n), lambda i: (0, 0))],
        out_specs=pl.BlockSpec((BM, n), lambda i: (i, 0)),
        out_shape=jax.ShapeDtypeStruct((m, n), jnp.float32),
    )(q0, q1, b2p)


def kernel(x, edge_index, edge_weight, W1, b1, W2, b2):
    src = edge_index[0].astype(jnp.int32)
    dst = edge_index[1].astype(jnp.int32)
    w = edge_weight.astype(jnp.float32)

    h1 = _tc_matmul(x, W1)
    p1 = _sc_agg(h1, src, dst, w, HIDDEN)

    w2p = jnp.pad(W2, ((0, 0), (0, DL - N_LABELS)))
    h2 = _tc_mid(p1[0], p1[1], b1.reshape(1, -1), w2p)
    p2 = _sc_agg(h2, src, dst, w, DL)

    b2p = jnp.pad(b2, (0, DL - N_LABELS)).reshape(1, -1)
    out = _tc_softmax(p2[0], p2[1], b2p)
    return out[:, :N_LABELS]


# speedup vs baseline: 15.8053x; 1.8867x over previous
import dataclasses
import functools

import jax
import jax.numpy as jnp
from jax import lax
from jax.experimental import pallas as pl
from jax.experimental.pallas import tpu as pltpu
from jax.experimental.pallas import tpu_sc as plsc

N_NODES = 10000
N_EDGES = 320000
D_FEAT = 128
HIDDEN = 64
DL = 16
N_LABELS = 3

NC = 2
NS = 16
NW = NC * NS
L = 16
EC = 128
NCHUNK = N_EDGES // EC
NFC = NCHUNK // NW
NEXTRA = NCHUNK - NFC * NW
RB = 80
NRB = N_NODES // RB

BM = 2000


def _sc_agg(h, src2d, dst2d, w2d, D):
    mesh = plsc.VectorSubcoreMesh(core_axis_name="c", subcore_axis_name="s")
    cp = pltpu.CompilerParams(use_tc_tiling_on_sc=False)
    if "needs_layout_passes" in pltpu.CompilerParams.__dataclass_fields__:
        cp = dataclasses.replace(cp, needs_layout_passes=False)

    @functools.partial(
        pl.kernel,
        mesh=mesh,
        compiler_params=cp,
        out_type=jax.ShapeDtypeStruct((NC, N_NODES, D), jnp.float32),
        scratch_types=[
            pltpu.VMEM((NFC + 1, EC), jnp.int32),
            pltpu.VMEM((NFC + 1, EC), jnp.int32),
            pltpu.VMEM((NFC + 1, EC), jnp.float32),
            pltpu.VMEM((3, EC, D), jnp.float32),
            pltpu.VMEM((RB, D), jnp.float32),
            pltpu.VMEM_SHARED((N_NODES, D), jnp.float32),
            pltpu.SemaphoreType.DMA,
            pltpu.SemaphoreType.DMA,
            pltpu.SemaphoreType.DMA,
            pltpu.SemaphoreType.DMA,
            pltpu.SemaphoreType.DMA,
            pltpu.SemaphoreType.DMA,
        ],
    )
    def k(h_hbm, src_hbm, dst_hbm, w_hbm, out_hbm,
          src_v, dst_v, w_v, rows_v, stage_v, acc_sh,
          g0, g1, g2, s0, s1, s2):
        cid = lax.axis_index("c")
        sid = lax.axis_index("s")
        wid = sid * NC + cid
        gsem = (g0, g1, g2)
        ssem = (s0, s1, s2)
        nfc = NFC + jnp.where(wid < NEXTRA, 1, 0)

        crow = wid * NFC
        pltpu.sync_copy(src_hbm.at[pl.ds(crow, NFC)], src_v.at[pl.ds(0, NFC)])
        pltpu.sync_copy(dst_hbm.at[pl.ds(crow, NFC)], dst_v.at[pl.ds(0, NFC)])
        pltpu.sync_copy(w_hbm.at[pl.ds(crow, NFC)], w_v.at[pl.ds(0, NFC)])

        @pl.when(wid < NEXTRA)
        def _():
            xrow = NFC * NW + wid
            pltpu.sync_copy(src_hbm.at[pl.ds(xrow, 1)], src_v.at[pl.ds(NFC, 1)])
            pltpu.sync_copy(dst_hbm.at[pl.ds(xrow, 1)], dst_v.at[pl.ds(NFC, 1)])
            pltpu.sync_copy(w_hbm.at[pl.ds(xrow, 1)], w_v.at[pl.ds(NFC, 1)])

        zvec = jnp.zeros((L,), jnp.float32)

        @pl.loop(0, RB)
        def _(i):
            for j in range(D // L):
                stage_v[i, pl.ds(j * L, L)] = zvec

        @pl.loop(sid, NRB, step=NS)
        def _(b):
            pltpu.sync_copy(stage_v, acc_sh.at[pl.ds(b * RB, RB)])

        plsc.subcore_barrier()

        def start_gather(c, b):
            pltpu.async_copy(h_hbm.at[src_v.at[c]], rows_v.at[b], gsem[b])

        def wait_gather(c, b):
            pltpu.make_async_copy(h_hbm.at[src_v.at[c]], rows_v.at[b],
                                  gsem[b]).wait()

        def start_scatter(c, b):
            pltpu.async_copy(rows_v.at[b], acc_sh.at[dst_v.at[c]], ssem[b],
                             add=True)

        def wait_scatter(c, b):
            pltpu.make_async_copy(rows_v.at[b], acc_sh.at[dst_v.at[c]],
                                  ssem[b]).wait()

        def scale(c, b):
            @plsc.parallel_loop(0, EC, unroll=8)
            def _(e):
                wb = plsc.load_gather(w_v.at[c], [jnp.full((L,), e, jnp.int32)])
                for j in range(D // L):
                    rows_v[b, e, pl.ds(j * L, L)] = (
                        rows_v[b, e, pl.ds(j * L, L)] * wb)

        def stage(c, bprev, bcur, bnext):
            @pl.when(c + 1 < nfc)
            def _():
                start_gather(c + 1, bnext)

            wait_gather(c, bcur)
            scale(c, bcur)
            start_scatter(c, bcur)

            @pl.when(c >= 1)
            def _():
                wait_scatter(c - 1, bprev)

        start_gather(0, 0)

        @pl.loop(0, NFC, step=3)
        def _(c):
            stage(c, 2, 0, 1)
            stage(c + 1, 0, 1, 2)
            stage(c + 2, 1, 2, 0)

        @pl.when(nfc > NFC)
        def _():
            stage(NFC, (NFC - 1) % 3, NFC % 3, (NFC + 1) % 3)

        @pl.when(nfc > NFC)
        def _():
            wait_scatter(NFC, NFC % 3)

        @pl.when(nfc == NFC)
        def _():
            wait_scatter(NFC - 1, (NFC - 1) % 3)

        plsc.subcore_barrier()

        @pl.loop(sid, NRB, step=NS)
        def _(b):
            pltpu.sync_copy(acc_sh.at[pl.ds(b * RB, RB)], stage_v)
            pltpu.sync_copy(stage_v, out_hbm.at[cid, pl.ds(b * RB, RB)])

    return k(h, src2d, dst2d, w2d)


def _mm_kernel(x_ref, w_ref, o_ref):
    o_ref[...] = jnp.dot(x_ref[...], w_ref[...],
                         preferred_element_type=jnp.float32,
                         precision=lax.Precision.HIGHEST)


def _tc_matmul(x, w):
    m, kdim = x.shape
    n = w.shape[1]
    return pl.pallas_call(
        _mm_kernel,
        grid=(m // BM,),
        in_specs=[pl.BlockSpec((BM, kdim), lambda i: (i, 0)),
                  pl.BlockSpec((kdim, n), lambda i: (0, 0))],
        out_specs=pl.BlockSpec((BM, n), lambda i: (i, 0)),
        out_shape=jax.ShapeDtypeStruct((m, n), jnp.float32),
    )(x, w)


def _mid_kernel(p0_ref, p1_ref, b1_ref, w2_ref, o_ref):
    h = jnp.maximum(p0_ref[...] + p1_ref[...] + b1_ref[...], 0.0)
    o_ref[...] = jnp.dot(h, w2_ref[...],
                         preferred_element_type=jnp.float32,
                         precision=lax.Precision.HIGHEST)


def _tc_mid(p0, p1, b1, w2p):
    m, kdim = p0.shape
    n = w2p.shape[1]
    return pl.pallas_call(
        _mid_kernel,
        grid=(m // BM,),
        in_specs=[pl.BlockSpec((BM, kdim), lambda i: (i, 0)),
                  pl.BlockSpec((BM, kdim), lambda i: (i, 0)),
                  pl.BlockSpec((1, kdim), lambda i: (0, 0)),
                  pl.BlockSpec((kdim, n), lambda i: (0, 0))],
        out_specs=pl.BlockSpec((BM, n), lambda i: (i, 0)),
        out_shape=jax.ShapeDtypeStruct((m, n), jnp.float32),
    )(p0, p1, b1, w2p)


def _sm_kernel(q0_ref, q1_ref, b2_ref, o_ref):
    s = q0_ref[...] + q1_ref[...] + b2_ref[...]
    col = lax.broadcasted_iota(jnp.int32, s.shape, 1)
    s = jnp.where(col < N_LABELS, s, -1e30)
    mx = jnp.max(s, axis=-1, keepdims=True)
    e = jnp.exp(s - mx)
    o_ref[...] = e / jnp.sum(e, axis=-1, keepdims=True)


def _tc_softmax(q0, q1, b2p):
    m, n = q0.shape
    return pl.pallas_call(
        _sm_kernel,
        grid=(m // BM,),
        in_specs=[pl.BlockSpec((BM, n), lambda i: (i, 0)),
                  pl.BlockSpec((BM, n), lambda i: (i, 0)),
                  pl.BlockSpec((1, n), lambda i: (0, 0))],
        out_specs=pl.BlockSpec((BM, n), lambda i: (i, 0)),
        out_shape=jax.ShapeDtypeStruct((m, n), jnp.float32),
    )(q0, q1, b2p)


def kernel(x, edge_index, edge_weight, W1, b1, W2, b2):
    src2d = edge_index[0].astype(jnp.int32).reshape(NCHUNK, EC)
    dst2d = edge_index[1].astype(jnp.int32).reshape(NCHUNK, EC)
    w2d = edge_weight.astype(jnp.float32).reshape(NCHUNK, EC)

    h1 = _tc_matmul(x, W1)
    p1 = _sc_agg(h1, src2d, dst2d, w2d, HIDDEN)

    w2p = jnp.pad(W2, ((0, 0), (0, DL - N_LABELS)))
    h2 = _tc_mid(p1[0], p1[1], b1.reshape(1, -1), w2p)
    p2 = _sc_agg(h2, src2d, dst2d, w2d, DL)

    b2p = jnp.pad(b2, (0, DL - N_LABELS)).reshape(1, -1)
    out = _tc_softmax(p2[0], p2[1], b2p)
    return out[:, :N_LABELS]


# speedup vs baseline: 16.0535x; 1.0157x over previous
import dataclasses
import functools

import jax
import jax.numpy as jnp
from jax import lax
from jax.experimental import pallas as pl
from jax.experimental.pallas import tpu as pltpu
from jax.experimental.pallas import tpu_sc as plsc

N_NODES = 10000
N_EDGES = 320000
D_FEAT = 128
HIDDEN = 64
DL = 16
N_LABELS = 3

NC = 2
NS = 16
NW = NC * NS
L = 16
EC = 128
NCHUNK = N_EDGES // EC
NFC = NCHUNK // NW
NEXTRA = NCHUNK - NFC * NW
RB = 80
NRB = N_NODES // RB

BM = 2000


def _sc_agg(h, src2d, dst2d, w2d, D):
    mesh = plsc.VectorSubcoreMesh(core_axis_name="c", subcore_axis_name="s")
    cp = pltpu.CompilerParams(use_tc_tiling_on_sc=False)
    if "needs_layout_passes" in pltpu.CompilerParams.__dataclass_fields__:
        cp = dataclasses.replace(cp, needs_layout_passes=False)

    @functools.partial(
        pl.kernel,
        mesh=mesh,
        compiler_params=cp,
        out_type=jax.ShapeDtypeStruct((NC, N_NODES, D), jnp.float32),
        scratch_types=[
            pltpu.VMEM((NFC + 1, EC), jnp.int32),
            pltpu.VMEM((NFC + 1, EC), jnp.int32),
            pltpu.VMEM((NFC + 1, EC), jnp.float32),
            pltpu.VMEM((3, EC, D), jnp.float32),
            pltpu.VMEM((RB, D), jnp.float32),
            pltpu.VMEM_SHARED((N_NODES, D), jnp.float32),
            pltpu.SemaphoreType.DMA,
            pltpu.SemaphoreType.DMA,
            pltpu.SemaphoreType.DMA,
            pltpu.SemaphoreType.DMA,
            pltpu.SemaphoreType.DMA,
            pltpu.SemaphoreType.DMA,
        ],
    )
    def k(h_hbm, src_hbm, dst_hbm, w_hbm, out_hbm,
          src_v, dst_v, w_v, rows_v, stage_v, acc_sh,
          g0, g1, g2, s0, s1, s2):
        cid = lax.axis_index("c")
        sid = lax.axis_index("s")
        wid = sid * NC + cid
        gsem = (g0, g1, g2)
        ssem = (s0, s1, s2)
        nfc = NFC + jnp.where(wid < NEXTRA, 1, 0)

        crow = wid * NFC
        pltpu.sync_copy(src_hbm.at[pl.ds(crow, NFC)], src_v.at[pl.ds(0, NFC)])
        pltpu.sync_copy(dst_hbm.at[pl.ds(crow, NFC)], dst_v.at[pl.ds(0, NFC)])
        pltpu.sync_copy(w_hbm.at[pl.ds(crow, NFC)], w_v.at[pl.ds(0, NFC)])

        @pl.when(wid < NEXTRA)
        def _():
            xrow = NFC * NW + wid
            pltpu.sync_copy(src_hbm.at[pl.ds(xrow, 1)], src_v.at[pl.ds(NFC, 1)])
            pltpu.sync_copy(dst_hbm.at[pl.ds(xrow, 1)], dst_v.at[pl.ds(NFC, 1)])
            pltpu.sync_copy(w_hbm.at[pl.ds(xrow, 1)], w_v.at[pl.ds(NFC, 1)])

        zvec = jnp.zeros((L,), jnp.float32)

        @pl.loop(0, RB)
        def _(i):
            for j in range(D // L):
                stage_v[i, pl.ds(j * L, L)] = zvec

        @pl.loop(sid, NRB, step=NS)
        def _(b):
            pltpu.sync_copy(stage_v, acc_sh.at[pl.ds(b * RB, RB)])

        plsc.subcore_barrier()

        def start_gather(c, b):
            pltpu.async_copy(h_hbm.at[src_v.at[c]], rows_v.at[b], gsem[b])

        def wait_gather(c, b):
            pltpu.make_async_copy(h_hbm.at[src_v.at[c]], rows_v.at[b],
                                  gsem[b]).wait()

        def start_scatter(c, b):
            pltpu.async_copy(rows_v.at[b], acc_sh.at[dst_v.at[c]], ssem[b],
                             add=True)

        def wait_scatter(c, b):
            pltpu.make_async_copy(rows_v.at[b], acc_sh.at[dst_v.at[c]],
                                  ssem[b]).wait()

        def scale(c, b):
            @plsc.parallel_loop(0, EC, unroll=8)
            def _(e):
                wb = plsc.load_gather(w_v.at[c], [jnp.full((L,), e, jnp.int32)])
                for j in range(D // L):
                    rows_v[b, e, pl.ds(j * L, L)] = (
                        rows_v[b, e, pl.ds(j * L, L)] * wb)

        def stage(c, bprev, bcur, bnext):
            @pl.when(c + 1 < nfc)
            def _():
                start_gather(c + 1, bnext)

            wait_gather(c, bcur)
            scale(c, bcur)
            start_scatter(c, bcur)

            @pl.when(c >= 1)
            def _():
                wait_scatter(c - 1, bprev)

        start_gather(0, 0)

        @pl.loop(0, NFC, step=3)
        def _(c):
            stage(c, 2, 0, 1)
            stage(c + 1, 0, 1, 2)
            stage(c + 2, 1, 2, 0)

        @pl.when(nfc > NFC)
        def _():
            stage(NFC, (NFC - 1) % 3, NFC % 3, (NFC + 1) % 3)

        @pl.when(nfc > NFC)
        def _():
            wait_scatter(NFC, NFC % 3)

        @pl.when(nfc == NFC)
        def _():
            wait_scatter(NFC - 1, (NFC - 1) % 3)

        plsc.subcore_barrier()

        @pl.loop(sid, NRB, step=NS)
        def _(b):
            pltpu.sync_copy(acc_sh.at[pl.ds(b * RB, RB)], stage_v)
            pltpu.sync_copy(stage_v, out_hbm.at[cid, pl.ds(b * RB, RB)])

    return k(h, src2d, dst2d, w2d)


def _mm_kernel(x_ref, w_ref, o_ref):
    o_ref[...] = jnp.dot(x_ref[...], w_ref[...],
                         preferred_element_type=jnp.float32,
                         precision=lax.Precision.DEFAULT)


def _tc_matmul(x, w):
    m, kdim = x.shape
    n = w.shape[1]
    return pl.pallas_call(
        _mm_kernel,
        grid=(m // BM,),
        in_specs=[pl.BlockSpec((BM, kdim), lambda i: (i, 0)),
                  pl.BlockSpec((kdim, n), lambda i: (0, 0))],
        out_specs=pl.BlockSpec((BM, n), lambda i: (i, 0)),
        out_shape=jax.ShapeDtypeStruct((m, n), jnp.float32),
    )(x, w)


def _mid_kernel(p0_ref, p1_ref, b1_ref, w2_ref, o_ref):
    h = jnp.maximum(p0_ref[...] + p1_ref[...] + b1_ref[...], 0.0)
    o_ref[...] = jnp.dot(h, w2_ref[...],
                         preferred_element_type=jnp.float32,
                         precision=lax.Precision.DEFAULT)


def _tc_mid(p0, p1, b1, w2p):
    m, kdim = p0.shape
    n = w2p.shape[1]
    return pl.pallas_call(
        _mid_kernel,
        grid=(m // BM,),
        in_specs=[pl.BlockSpec((BM, kdim), lambda i: (i, 0)),
                  pl.BlockSpec((BM, kdim), lambda i: (i, 0)),
                  pl.BlockSpec((1, kdim), lambda i: (0, 0)),
                  pl.BlockSpec((kdim, n), lambda i: (0, 0))],
        out_specs=pl.BlockSpec((BM, n), lambda i: (i, 0)),
        out_shape=jax.ShapeDtypeStruct((m, n), jnp.float32),
    )(p0, p1, b1, w2p)


def _sm_kernel(q0_ref, q1_ref, b2_ref, o_ref):
    s = q0_ref[...] + q1_ref[...] + b2_ref[...]
    col = lax.broadcasted_iota(jnp.int32, s.shape, 1)
    s = jnp.where(col < N_LABELS, s, -1e30)
    mx = jnp.max(s, axis=-1, keepdims=True)
    e = jnp.exp(s - mx)
    o_ref[...] = e / jnp.sum(e, axis=-1, keepdims=True)


def _tc_softmax(q0, q1, b2p):
    m, n = q0.shape
    return pl.pallas_call(
        _sm_kernel,
        grid=(m // BM,),
        in_specs=[pl.BlockSpec((BM, n), lambda i: (i, 0)),
                  pl.BlockSpec((BM, n), lambda i: (i, 0)),
                  pl.BlockSpec((1, n), lambda i: (0, 0))],
        out_specs=pl.BlockSpec((BM, n), lambda i: (i, 0)),
        out_shape=jax.ShapeDtypeStruct((m, n), jnp.float32),
    )(q0, q1, b2p)


def kernel(x, edge_index, edge_weight, W1, b1, W2, b2):
    src2d = edge_index[0].astype(jnp.int32).reshape(NCHUNK, EC)
    dst2d = edge_index[1].astype(jnp.int32).reshape(NCHUNK, EC)
    w2d = edge_weight.astype(jnp.float32).reshape(NCHUNK, EC)

    h1 = _tc_matmul(x, W1)
    p1 = _sc_agg(h1, src2d, dst2d, w2d, HIDDEN)

    w2p = jnp.pad(W2, ((0, 0), (0, DL - N_LABELS)))
    h2 = _tc_mid(p1[0], p1[1], b1.reshape(1, -1), w2p)
    p2 = _sc_agg(h2, src2d, dst2d, w2d, DL)

    b2p = jnp.pad(b2, (0, DL - N_LABELS)).reshape(1, -1)
    out = _tc_softmax(p2[0], p2[1], b2p)
    return out[:, :N_LABELS]


# speedup vs baseline: 18.0539x; 1.1246x over previous
import dataclasses
import functools

import jax
import jax.numpy as jnp
from jax import lax
from jax.experimental import pallas as pl
from jax.experimental.pallas import tpu as pltpu
from jax.experimental.pallas import tpu_sc as plsc

N_NODES = 10000
N_EDGES = 320000
D_FEAT = 128
HIDDEN = 64
DL = 16
N_LABELS = 3

NC = 2
NS = 16
NW = NC * NS
L = 16
EC = 128
NCHUNK = N_EDGES // EC
NFC = NCHUNK // NW
NEXTRA = NCHUNK - NFC * NW
RB = 80
NRB = N_NODES // RB

BM = 2000


def _sc_agg(h, ei3, w2d, D):
    mesh = plsc.VectorSubcoreMesh(core_axis_name="c", subcore_axis_name="s")
    cp = pltpu.CompilerParams(use_tc_tiling_on_sc=False)
    if "needs_layout_passes" in pltpu.CompilerParams.__dataclass_fields__:
        cp = dataclasses.replace(cp, needs_layout_passes=False)

    @functools.partial(
        pl.kernel,
        mesh=mesh,
        compiler_params=cp,
        out_type=jax.ShapeDtypeStruct((NC, N_NODES, D), jnp.float32),
        scratch_types=[
            pltpu.VMEM((NFC + 1, EC), jnp.int32),
            pltpu.VMEM((NFC + 1, EC), jnp.int32),
            pltpu.VMEM((NFC + 1, EC), jnp.float32),
            pltpu.VMEM((3, EC, D), jnp.float32),
            pltpu.VMEM((RB, D), jnp.float32),
            pltpu.VMEM_SHARED((N_NODES, D), jnp.float32),
            pltpu.SemaphoreType.DMA,
            pltpu.SemaphoreType.DMA,
            pltpu.SemaphoreType.DMA,
            pltpu.SemaphoreType.DMA,
            pltpu.SemaphoreType.DMA,
            pltpu.SemaphoreType.DMA,
        ],
    )
    def k(h_hbm, ei_hbm, w_hbm, out_hbm,
          src_v, dst_v, w_v, rows_v, stage_v, acc_sh,
          g0, g1, g2, s0, s1, s2):
        cid = lax.axis_index("c")
        sid = lax.axis_index("s")
        wid = sid * NC + cid
        gsem = (g0, g1, g2)
        ssem = (s0, s1, s2)
        nfc = NFC + jnp.where(wid < NEXTRA, 1, 0)

        crow = wid * NFC
        pltpu.sync_copy(ei_hbm.at[0, pl.ds(crow, NFC)], src_v.at[pl.ds(0, NFC)])
        pltpu.sync_copy(ei_hbm.at[1, pl.ds(crow, NFC)], dst_v.at[pl.ds(0, NFC)])
        pltpu.sync_copy(w_hbm.at[pl.ds(crow, NFC)], w_v.at[pl.ds(0, NFC)])

        @pl.when(wid < NEXTRA)
        def _():
            xrow = NFC * NW + wid
            pltpu.sync_copy(ei_hbm.at[0, pl.ds(xrow, 1)],
                            src_v.at[pl.ds(NFC, 1)])
            pltpu.sync_copy(ei_hbm.at[1, pl.ds(xrow, 1)],
                            dst_v.at[pl.ds(NFC, 1)])
            pltpu.sync_copy(w_hbm.at[pl.ds(xrow, 1)], w_v.at[pl.ds(NFC, 1)])

        zvec = jnp.zeros((L,), jnp.float32)

        @pl.loop(0, RB)
        def _(i):
            for j in range(D // L):
                stage_v[i, pl.ds(j * L, L)] = zvec

        @pl.loop(sid, NRB, step=NS)
        def _(b):
            pltpu.sync_copy(stage_v, acc_sh.at[pl.ds(b * RB, RB)])

        plsc.subcore_barrier()

        def start_gather(c, b):
            pltpu.async_copy(h_hbm.at[src_v.at[c]], rows_v.at[b], gsem[b])

        def wait_gather(c, b):
            pltpu.make_async_copy(h_hbm.at[src_v.at[c]], rows_v.at[b],
                                  gsem[b]).wait()

        def start_scatter(c, b):
            pltpu.async_copy(rows_v.at[b], acc_sh.at[dst_v.at[c]], ssem[b],
                             add=True)

        def wait_scatter(c, b):
            pltpu.make_async_copy(rows_v.at[b], acc_sh.at[dst_v.at[c]],
                                  ssem[b]).wait()

        def scale(c, b):
            @plsc.parallel_loop(0, EC, unroll=8)
            def _(e):
                wb = plsc.load_gather(w_v.at[c], [jnp.full((L,), e, jnp.int32)])
                for j in range(D // L):
                    rows_v[b, e, pl.ds(j * L, L)] = (
                        rows_v[b, e, pl.ds(j * L, L)] * wb)

        def stage(c, bprev, bcur, bnext):
            @pl.when(c + 1 < nfc)
            def _():
                start_gather(c + 1, bnext)

            wait_gather(c, bcur)
            scale(c, bcur)
            start_scatter(c, bcur)

            @pl.when(c >= 1)
            def _():
                wait_scatter(c - 1, bprev)

        start_gather(0, 0)

        @pl.loop(0, NFC, step=3)
        def _(c):
            stage(c, 2, 0, 1)
            stage(c + 1, 0, 1, 2)
            stage(c + 2, 1, 2, 0)

        @pl.when(nfc > NFC)
        def _():
            stage(NFC, (NFC - 1) % 3, NFC % 3, (NFC + 1) % 3)

        @pl.when(nfc > NFC)
        def _():
            wait_scatter(NFC, NFC % 3)

        @pl.when(nfc == NFC)
        def _():
            wait_scatter(NFC - 1, (NFC - 1) % 3)

        plsc.subcore_barrier()

        @pl.loop(sid, NRB, step=NS)
        def _(b):
            pltpu.sync_copy(acc_sh.at[pl.ds(b * RB, RB)], stage_v)
            pltpu.sync_copy(stage_v, out_hbm.at[cid, pl.ds(b * RB, RB)])

    return k(h, ei3, w2d)


def _mm_kernel(x_ref, w_ref, o_ref):
    o_ref[...] = jnp.dot(x_ref[...], w_ref[...],
                         preferred_element_type=jnp.float32,
                         precision=lax.Precision.DEFAULT)


def _tc_matmul(x, w):
    m, kdim = x.shape
    n = w.shape[1]
    return pl.pallas_call(
        _mm_kernel,
        grid=(m // BM,),
        in_specs=[pl.BlockSpec((BM, kdim), lambda i: (i, 0)),
                  pl.BlockSpec((kdim, n), lambda i: (0, 0))],
        out_specs=pl.BlockSpec((BM, n), lambda i: (i, 0)),
        out_shape=jax.ShapeDtypeStruct((m, n), jnp.float32),
    )(x, w)


def _mid_kernel(p_ref, b1_ref, w2_ref, o_ref):
    h = jnp.maximum(p_ref[0] + p_ref[1] + b1_ref[...], 0.0)
    o_ref[...] = jnp.dot(h, w2_ref[...],
                         preferred_element_type=jnp.float32,
                         precision=lax.Precision.DEFAULT)


def _tc_mid(p, b1, w2p):
    _, m, kdim = p.shape
    n = w2p.shape[1]
    return pl.pallas_call(
        _mid_kernel,
        grid=(m // BM,),
        in_specs=[pl.BlockSpec((2, BM, kdim), lambda i: (0, i, 0)),
                  pl.BlockSpec((1, kdim), lambda i: (0, 0)),
                  pl.BlockSpec((kdim, n), lambda i: (0, 0))],
        out_specs=pl.BlockSpec((BM, n), lambda i: (i, 0)),
        out_shape=jax.ShapeDtypeStruct((m, n), jnp.float32),
    )(p, b1, w2p)


def _sm_kernel(q_ref, b2_ref, o_ref):
    s = q_ref[0] + q_ref[1] + b2_ref[...]
    col = lax.broadcasted_iota(jnp.int32, s.shape, 1)
    s = jnp.where(col < N_LABELS, s, -1e30)
    mx = jnp.max(s, axis=-1, keepdims=True)
    e = jnp.exp(s - mx)
    o_ref[...] = e / jnp.sum(e, axis=-1, keepdims=True)


def _tc_softmax(q, b2p):
    _, m, n = q.shape
    return pl.pallas_call(
        _sm_kernel,
        grid=(m // BM,),
        in_specs=[pl.BlockSpec((2, BM, n), lambda i: (0, i, 0)),
                  pl.BlockSpec((1, n), lambda i: (0, 0))],
        out_specs=pl.BlockSpec((BM, n), lambda i: (i, 0)),
        out_shape=jax.ShapeDtypeStruct((m, n), jnp.float32),
    )(q, b2p)


def kernel(x, edge_index, edge_weight, W1, b1, W2, b2):
    ei3 = edge_index.astype(jnp.int32).reshape(2, NCHUNK, EC)
    w2d = edge_weight.astype(jnp.float32).reshape(NCHUNK, EC)

    h1 = _tc_matmul(x, W1)
    p1 = _sc_agg(h1, ei3, w2d, HIDDEN)

    w2p = jnp.pad(W2, ((0, 0), (0, DL - N_LABELS)))
    h2 = _tc_mid(p1, b1.reshape(1, -1), w2p)
    p2 = _sc_agg(h2, ei3, w2d, DL)

    b2p = jnp.pad(b2, (0, DL - N_LABELS)).reshape(1, -1)
    out = _tc_softmax(p2, b2p)
    return out[:, :N_LABELS]


# speedup vs baseline: 19.5912x; 1.0852x over previous
import dataclasses
import functools

import jax
import jax.numpy as jnp
from jax import lax
from jax.experimental import pallas as pl
from jax.experimental.pallas import tpu as pltpu
from jax.experimental.pallas import tpu_sc as plsc

N_NODES = 10000
N_EDGES = 320000
D_FEAT = 128
HIDDEN = 64
DL = 16
N_LABELS = 3

NC = 2
NS = 16
NW = NC * NS
L = 16
EC = 128
NCHUNK = N_EDGES // EC
NFC = NCHUNK // NW
NEXTRA = NCHUNK - NFC * NW
RB = 80
NRB = N_NODES // RB

BM = 2000


def _sc_agg(h, ei3, w2d, D):
    mesh = plsc.VectorSubcoreMesh(core_axis_name="c", subcore_axis_name="s")
    cp = pltpu.CompilerParams(use_tc_tiling_on_sc=False)
    if "needs_layout_passes" in pltpu.CompilerParams.__dataclass_fields__:
        cp = dataclasses.replace(cp, needs_layout_passes=False)

    @functools.partial(
        pl.kernel,
        mesh=mesh,
        compiler_params=cp,
        out_type=jax.ShapeDtypeStruct((NC, N_NODES, D), jnp.float32),
        scratch_types=[
            pltpu.VMEM((NFC + 1, EC), jnp.int32),
            pltpu.VMEM((NFC + 1, EC), jnp.int32),
            pltpu.VMEM((NFC + 1, EC), jnp.float32),
            pltpu.VMEM((3, 2, EC, D), jnp.float32),
            pltpu.VMEM((RB, D), jnp.float32),
            pltpu.VMEM_SHARED((N_NODES, D), jnp.float32),
            pltpu.SemaphoreType.DMA,
            pltpu.SemaphoreType.DMA,
            pltpu.SemaphoreType.DMA,
            pltpu.SemaphoreType.DMA,
            pltpu.SemaphoreType.DMA,
            pltpu.SemaphoreType.DMA,
        ],
    )
    def k(h_hbm, ei_hbm, w_hbm, out_hbm,
          src_v, dst_v, w_v, rows_v, stage_v, acc_sh,
          g0, g1, g2, s0, s1, s2):
        cid = lax.axis_index("c")
        sid = lax.axis_index("s")
        wid = sid * NC + cid
        gsem = (g0, g1, g2)
        ssem = (s0, s1, s2)
        nfc = NFC + jnp.where(wid < NEXTRA, 1, 0)

        crow = wid * NFC
        pltpu.sync_copy(ei_hbm.at[0, pl.ds(crow, NFC)], src_v.at[pl.ds(0, NFC)])
        pltpu.sync_copy(ei_hbm.at[1, pl.ds(crow, NFC)], dst_v.at[pl.ds(0, NFC)])
        pltpu.sync_copy(w_hbm.at[pl.ds(crow, NFC)], w_v.at[pl.ds(0, NFC)])

        @pl.when(wid < NEXTRA)
        def _():
            xrow = NFC * NW + wid
            pltpu.sync_copy(ei_hbm.at[0, pl.ds(xrow, 1)],
                            src_v.at[pl.ds(NFC, 1)])
            pltpu.sync_copy(ei_hbm.at[1, pl.ds(xrow, 1)],
                            dst_v.at[pl.ds(NFC, 1)])
            pltpu.sync_copy(w_hbm.at[pl.ds(xrow, 1)], w_v.at[pl.ds(NFC, 1)])

        zvec = jnp.zeros((L,), jnp.float32)

        @pl.loop(0, RB)
        def _(i):
            for j in range(D // L):
                stage_v[i, pl.ds(j * L, L)] = zvec

        @pl.loop(sid, NRB, step=NS)
        def _(b):
            pltpu.sync_copy(stage_v, acc_sh.at[pl.ds(b * RB, RB)])

        plsc.subcore_barrier()

        def start_gather(c, b, u):
            pltpu.async_copy(h_hbm.at[src_v.at[c]], rows_v.at[b, u], gsem[b])

        def wait_gather(c, b, u):
            pltpu.make_async_copy(h_hbm.at[src_v.at[c]], rows_v.at[b, u],
                                  gsem[b]).wait()

        def start_scatter(c, b, u):
            pltpu.async_copy(rows_v.at[b, u], acc_sh.at[dst_v.at[c]], ssem[b],
                             add=True)

        def wait_scatter(c, b, u):
            pltpu.make_async_copy(rows_v.at[b, u], acc_sh.at[dst_v.at[c]],
                                  ssem[b]).wait()

        def scale(c, b, u):
            @plsc.parallel_loop(0, EC, unroll=8)
            def _(e):
                wb = plsc.load_gather(w_v.at[c], [jnp.full((L,), e, jnp.int32)])
                for j in range(D // L):
                    rows_v[b, u, e, pl.ds(j * L, L)] = (
                        rows_v[b, u, e, pl.ds(j * L, L)] * wb)

        NSS = NFC // 2

        def sstage(s, bprev, bcur, bnext):
            @pl.when(s + 1 < NSS)
            def _():
                start_gather(2 * s + 2, bnext, 0)
                start_gather(2 * s + 3, bnext, 1)

            wait_gather(2 * s, bcur, 0)
            scale(2 * s, bcur, 0)
            start_scatter(2 * s, bcur, 0)
            wait_gather(2 * s + 1, bcur, 1)
            scale(2 * s + 1, bcur, 1)
            start_scatter(2 * s + 1, bcur, 1)

            @pl.when(s >= 1)
            def _():
                wait_scatter(2 * s - 2, bprev, 0)
                wait_scatter(2 * s - 1, bprev, 1)

        start_gather(0, 0, 0)
        start_gather(1, 0, 1)

        @pl.loop(0, NSS, step=3)
        def _(s):
            sstage(s, 2, 0, 1)
            sstage(s + 1, 0, 1, 2)
            sstage(s + 2, 1, 2, 0)

        wait_scatter(NFC - 2, (NSS - 1) % 3, 0)
        wait_scatter(NFC - 1, (NSS - 1) % 3, 1)

        @pl.when(nfc > NFC)
        def _():
            start_gather(NFC, 0, 0)
            wait_gather(NFC, 0, 0)
            scale(NFC, 0, 0)
            start_scatter(NFC, 0, 0)
            wait_scatter(NFC, 0, 0)

        plsc.subcore_barrier()

        @pl.loop(sid, NRB, step=NS)
        def _(b):
            pltpu.sync_copy(acc_sh.at[pl.ds(b * RB, RB)], stage_v)
            pltpu.sync_copy(stage_v, out_hbm.at[cid, pl.ds(b * RB, RB)])

    return k(h, ei3, w2d)


def _mm_kernel(x_ref, w_ref, o_ref):
    o_ref[...] = jnp.dot(x_ref[...], w_ref[...],
                         preferred_element_type=jnp.float32,
                         precision=lax.Precision.DEFAULT)


def _tc_matmul(x, w):
    m, kdim = x.shape
    n = w.shape[1]
    return pl.pallas_call(
        _mm_kernel,
        grid=(m // BM,),
        in_specs=[pl.BlockSpec((BM, kdim), lambda i: (i, 0)),
                  pl.BlockSpec((kdim, n), lambda i: (0, 0))],
        out_specs=pl.BlockSpec((BM, n), lambda i: (i, 0)),
        out_shape=jax.ShapeDtypeStruct((m, n), jnp.float32),
    )(x, w)


def _mid_kernel(p_ref, b1_ref, w2_ref, o_ref):
    h = jnp.maximum(p_ref[0] + p_ref[1] + b1_ref[...], 0.0)
    o_ref[...] = jnp.dot(h, w2_ref[...],
                         preferred_element_type=jnp.float32,
                         precision=lax.Precision.DEFAULT)


def _tc_mid(p, b1, w2p):
    _, m, kdim = p.shape
    n = w2p.shape[1]
    return pl.pallas_call(
        _mid_kernel,
        grid=(m // BM,),
        in_specs=[pl.BlockSpec((2, BM, kdim), lambda i: (0, i, 0)),
                  pl.BlockSpec((1, kdim), lambda i: (0, 0)),
                  pl.BlockSpec((kdim, n), lambda i: (0, 0))],
        out_specs=pl.BlockSpec((BM, n), lambda i: (i, 0)),
        out_shape=jax.ShapeDtypeStruct((m, n), jnp.float32),
    )(p, b1, w2p)


def _sm_kernel(q_ref, b2_ref, o_ref):
    s = q_ref[0] + q_ref[1] + b2_ref[...]
    col = lax.broadcasted_iota(jnp.int32, s.shape, 1)
    s = jnp.where(col < N_LABELS, s, -1e30)
    mx = jnp.max(s, axis=-1, keepdims=True)
    e = jnp.exp(s - mx)
    o_ref[...] = e / jnp.sum(e, axis=-1, keepdims=True)


def _tc_softmax(q, b2p):
    _, m, n = q.shape
    return pl.pallas_call(
        _sm_kernel,
        grid=(m // BM,),
        in_specs=[pl.BlockSpec((2, BM, n), lambda i: (0, i, 0)),
                  pl.BlockSpec((1, n), lambda i: (0, 0))],
        out_specs=pl.BlockSpec((BM, n), lambda i: (i, 0)),
        out_shape=jax.ShapeDtypeStruct((m, n), jnp.float32),
    )(q, b2p)


def kernel(x, edge_index, edge_weight, W1, b1, W2, b2):
    ei3 = edge_index.astype(jnp.int32).reshape(2, NCHUNK, EC)
    w2d = edge_weight.astype(jnp.float32).reshape(NCHUNK, EC)

    h1 = _tc_matmul(x, W1)
    p1 = _sc_agg(h1, ei3, w2d, HIDDEN)

    w2p = jnp.pad(W2, ((0, 0), (0, DL - N_LABELS)))
    h2 = _tc_mid(p1, b1.reshape(1, -1), w2p)
    p2 = _sc_agg(h2, ei3, w2d, DL)

    b2p = jnp.pad(b2, (0, DL - N_LABELS)).reshape(1, -1)
    out = _tc_softmax(p2, b2p)
    return out[:, :N_LABELS]


# speedup vs baseline: 22.1063x; 1.1284x over previous
import dataclasses
import functools

import jax
import jax.numpy as jnp
from jax import lax
from jax.experimental import pallas as pl
from jax.experimental.pallas import tpu as pltpu
from jax.experimental.pallas import tpu_sc as plsc

N_NODES = 10000
N_EDGES = 320000
D_FEAT = 128
HIDDEN = 64
DL = 16
N_LABELS = 3

NC = 2
NS = 16
NW = NC * NS
L = 16
EC = 128
NCHUNK = N_EDGES // EC
NFC = NCHUNK // NW
NEXTRA = NCHUNK - NFC * NW
RB = 80
NRB = N_NODES // RB

BM = 2000


def _sc_agg(h, ei3, w2d, D):
    mesh = plsc.VectorSubcoreMesh(core_axis_name="c", subcore_axis_name="s")
    cp = pltpu.CompilerParams(use_tc_tiling_on_sc=False)
    if "needs_layout_passes" in pltpu.CompilerParams.__dataclass_fields__:
        cp = dataclasses.replace(cp, needs_layout_passes=False)

    @functools.partial(
        pl.kernel,
        mesh=mesh,
        compiler_params=cp,
        out_type=jax.ShapeDtypeStruct((NC, N_NODES, D), jnp.float32),
        scratch_types=[
            pltpu.VMEM((NFC + 1, EC), jnp.int32),
            pltpu.VMEM((NFC + 1, EC), jnp.int32),
            pltpu.VMEM((NFC + 1, EC), jnp.float32),
            pltpu.VMEM((3, 2, EC, D), jnp.float32),
            pltpu.VMEM((RB, D), jnp.float32),
            pltpu.VMEM_SHARED((N_NODES, D), jnp.float32),
            pltpu.SemaphoreType.DMA,
            pltpu.SemaphoreType.DMA,
            pltpu.SemaphoreType.DMA,
            pltpu.SemaphoreType.DMA,
            pltpu.SemaphoreType.DMA,
            pltpu.SemaphoreType.DMA,
        ],
    )
    def k(h_hbm, ei_hbm, w_hbm, out_hbm,
          src_v, dst_v, w_v, rows_v, stage_v, acc_sh,
          g0, g1, g2, s0, s1, s2):
        cid = lax.axis_index("c")
        sid = lax.axis_index("s")
        wid = sid * NC + cid
        gsem = (g0, g1, g2)
        ssem = (s0, s1, s2)
        nfc = NFC + jnp.where(wid < NEXTRA, 1, 0)

        crow = wid * NFC
        pltpu.sync_copy(ei_hbm.at[0, pl.ds(crow, NFC)], src_v.at[pl.ds(0, NFC)])
        pltpu.sync_copy(ei_hbm.at[1, pl.ds(crow, NFC)], dst_v.at[pl.ds(0, NFC)])
        pltpu.sync_copy(w_hbm.at[pl.ds(crow, NFC)], w_v.at[pl.ds(0, NFC)])

        @pl.when(wid < NEXTRA)
        def _():
            xrow = NFC * NW + wid
            pltpu.sync_copy(ei_hbm.at[0, pl.ds(xrow, 1)],
                            src_v.at[pl.ds(NFC, 1)])
            pltpu.sync_copy(ei_hbm.at[1, pl.ds(xrow, 1)],
                            dst_v.at[pl.ds(NFC, 1)])
            pltpu.sync_copy(w_hbm.at[pl.ds(xrow, 1)], w_v.at[pl.ds(NFC, 1)])

        zvec = jnp.zeros((L,), jnp.float32)

        @pl.loop(0, RB)
        def _(i):
            for j in range(D // L):
                stage_v[i, pl.ds(j * L, L)] = zvec

        @pl.loop(sid, NRB, step=NS)
        def _(b):
            pltpu.sync_copy(stage_v, acc_sh.at[pl.ds(b * RB, RB)])

        plsc.subcore_barrier()

        def start_gather(c, b, u):
            pltpu.async_copy(h_hbm.at[src_v.at[c]], rows_v.at[b, u], gsem[b])

        def wait_gather(c, b, u):
            pltpu.make_async_copy(h_hbm.at[src_v.at[c]], rows_v.at[b, u],
                                  gsem[b]).wait()

        def start_scatter(c, b, u):
            pltpu.async_copy(rows_v.at[b, u], acc_sh.at[dst_v.at[c]], ssem[b],
                             add=True)

        def wait_scatter(c, b, u):
            pltpu.make_async_copy(rows_v.at[b, u], acc_sh.at[dst_v.at[c]],
                                  ssem[b]).wait()

        def scale(c, b, u):
            @plsc.parallel_loop(0, EC, unroll=8)
            def _(e):
                wb = plsc.load_gather(w_v.at[c], [jnp.full((L,), e, jnp.int32)])
                for j in range(D // L):
                    rows_v[b, u, e, pl.ds(j * L, L)] = (
                        rows_v[b, u, e, pl.ds(j * L, L)] * wb)

        NSS = NFC // 2

        def sstage(s, bprev, bcur, bnext):
            @pl.when(s + 1 < NSS)
            def _():
                start_gather(2 * s + 2, bnext, 0)
                start_gather(2 * s + 3, bnext, 1)

            wait_gather(2 * s, bcur, 0)
            scale(2 * s, bcur, 0)
            start_scatter(2 * s, bcur, 0)
            wait_gather(2 * s + 1, bcur, 1)
            scale(2 * s + 1, bcur, 1)
            start_scatter(2 * s + 1, bcur, 1)

            @pl.when(s >= 1)
            def _():
                wait_scatter(2 * s - 2, bprev, 0)
                wait_scatter(2 * s - 1, bprev, 1)

        start_gather(0, 0, 0)
        start_gather(1, 0, 1)

        @pl.loop(0, NSS, step=3)
        def _(s):
            sstage(s, 2, 0, 1)
            sstage(s + 1, 0, 1, 2)
            sstage(s + 2, 1, 2, 0)

        wait_scatter(NFC - 2, (NSS - 1) % 3, 0)
        wait_scatter(NFC - 1, (NSS - 1) % 3, 1)

        @pl.when(nfc > NFC)
        def _():
            start_gather(NFC, 0, 0)
            wait_gather(NFC, 0, 0)
            scale(NFC, 0, 0)
            start_scatter(NFC, 0, 0)
            wait_scatter(NFC, 0, 0)

        plsc.subcore_barrier()

        @pl.loop(sid, NRB, step=NS)
        def _(b):
            pltpu.sync_copy(acc_sh.at[pl.ds(b * RB, RB)], stage_v)
            pltpu.sync_copy(stage_v, out_hbm.at[cid, pl.ds(b * RB, RB)])

    return k(h, ei3, w2d)


def _mm_kernel(x_ref, w_ref, o_ref):
    o_ref[...] = jnp.dot(x_ref[...], w_ref[...],
                         preferred_element_type=jnp.float32,
                         precision=lax.Precision.DEFAULT)


def _tc_matmul(x, w):
    m, kdim = x.shape
    n = w.shape[1]
    return pl.pallas_call(
        _mm_kernel,
        grid=(m // BM,),
        in_specs=[pl.BlockSpec((BM, kdim), lambda i: (i, 0)),
                  pl.BlockSpec((kdim, n), lambda i: (0, 0))],
        out_specs=pl.BlockSpec((BM, n), lambda i: (i, 0)),
        out_shape=jax.ShapeDtypeStruct((m, n), jnp.float32),
    )(x, w)


def _mid_kernel(p_ref, b1_ref, wd_ref, o_ref):
    h = jnp.maximum(p_ref[0] + p_ref[1] + b1_ref[...], 0.0)
    o_ref[...] = jnp.dot(h, wd_ref[...],
                         preferred_element_type=jnp.float32,
                         precision=lax.Precision.DEFAULT)


def _tc_mid(p128, b1t, wd):
    return pl.pallas_call(
        _mid_kernel,
        out_shape=jax.ShapeDtypeStruct((N_NODES // 2, 2 * DL), jnp.float32),
    )(p128, b1t, wd)


def _sm_kernel(q_ref, b2_ref, g_ref, gt_ref, o_ref):
    s = q_ref[0] + q_ref[1] + b2_ref[...]
    col = lax.broadcasted_iota(jnp.int32, s.shape, 1)
    s = jnp.where(col % DL < N_LABELS, s, -1e30)
    mx = jnp.max(s, axis=-1, keepdims=True)
    e = jnp.exp(s - mx)
    gsum = jnp.dot(e, g_ref[...], preferred_element_type=jnp.float32,
                   precision=lax.Precision.HIGHEST)
    denom = jnp.dot(gsum, gt_ref[...], preferred_element_type=jnp.float32,
                    precision=lax.Precision.HIGHEST)
    o_ref[...] = e / denom


def _tc_softmax(q128, b2t, g, gt):
    return pl.pallas_call(
        _sm_kernel,
        out_shape=jax.ShapeDtypeStruct((N_NODES * DL // 128, 128),
                                       jnp.float32),
    )(q128, b2t, g, gt)


def kernel(x, edge_index, edge_weight, W1, b1, W2, b2):
    ei3 = edge_index.astype(jnp.int32).reshape(2, NCHUNK, EC)
    w2d = edge_weight.astype(jnp.float32).reshape(NCHUNK, EC)

    h1 = _tc_matmul(x, W1)
    p1 = _sc_agg(h1, ei3, w2d, HIDDEN)

    w2p = jnp.pad(W2, ((0, 0), (0, DL - N_LABELS)))
    wd = jnp.kron(jnp.eye(2, dtype=jnp.float32), w2p)
    b1t = jnp.tile(b1, 2).reshape(1, 2 * HIDDEN)
    h2 = _tc_mid(p1.reshape(2, N_NODES // 2, 2 * HIDDEN), b1t, wd)
    p2 = _sc_agg(h2.reshape(N_NODES, DL), ei3, w2d, DL)

    b2p = jnp.pad(b2, (0, DL - N_LABELS))
    b2t = jnp.tile(b2p, 128 // DL).reshape(1, 128)
    g = jnp.kron(jnp.eye(128 // DL, dtype=jnp.float32),
                 jnp.ones((DL, 1), jnp.float32))
    out = _tc_softmax(p2.reshape(2, N_NODES * DL // 128, 128),
                      b2t, g, g.T)
    return out.reshape(N_NODES, DL)[:, :N_LABELS]
